# Initial kernel scaffold; baseline (speedup 1.0000x reference)
import jax
import jax.numpy as jnp
from jax.experimental import pallas as pl


def kernel(x, y, wc1, wb1, wc2, wb2, wc3, wb3, wc4, wb4, wfc, wfcb, sc1, sb1, sc2, sb2, sc3, sb3, sfc, sfcb, g0s, g0n, g0b, g1s, g1n, g1b, lih, lhh, lbi, lbh, r1, r1b, r2, r2b, edge_src0, edge_dst0, edge_src1, edge_dst1):
    raise NotImplementedError("write your pallas kernel here")


# speedup vs baseline: 3.4971x; 3.4971x over previous
import jax
import jax.numpy as jnp
import numpy as np
from jax import lax
from jax.experimental import pallas as pl
from jax.experimental.pallas import tpu as pltpu
from jax.experimental.pallas import tpu_sc as plsc

N0, N1, N2 = 50176, 7168, 1024
T = 5
H = 128

_R = 256
_NB = N0 // _R
_RB = 896
_NW = 32

_E0, _NC0, _CH0 = 43008, 12, 112
_E1, _NC1, _CH1 = 6144, 2, 96
_W0 = 128


def _conv_matrix(w, l_in, l_out):
    c_out, c_in, k = w.shape
    rows, cols, sel = [], [], []
    for co in range(c_out):
        for ci in range(c_in):
            for kk in range(k):
                for lo in range(l_out):
                    rows.append(ci * l_in + lo + kk)
                    cols.append(co * l_out + lo)
                    sel.append((co * c_in + ci) * k + kk)
    m = jnp.zeros((c_in * l_in, c_out * l_out), jnp.float32)
    return m.at[np.array(rows), np.array(cols)].add(w.reshape(-1)[np.array(sel)])


def _pool_matrix(c, l_in):
    l_out = l_in // 2
    p = np.zeros((c * l_in, c * l_out), np.float32)
    for ci in range(c):
        for l in range(l_in):
            p[ci * l_in + l, ci * l_out + l // 2] = 0.5
    return jnp.asarray(p)


def _cnn_body(x_ref, m1, b1, m2, b2, m3, b3, m4, b4, fw, bfw,
              s1, c1, s2, c2, s3, c3, fs, bfs, out_ref):
    f32 = jnp.float32
    for t in range(T):
        xb = x_ref[:, t, :]

        xw = jnp.concatenate([xb[:, s * 52:(s + 1) * 52] for s in range(6)],
                             axis=0)
        h = jnp.maximum(jnp.dot(xw, m1[...], preferred_element_type=f32) + b1[...], 0.0)
        h = jnp.maximum(jnp.dot(h, m2[...], preferred_element_type=f32) + b2[...], 0.0)
        h = jnp.maximum(jnp.dot(h, m3[...], preferred_element_type=f32) + b3[...], 0.0)
        h = jnp.maximum(jnp.dot(h, m4[...], preferred_element_type=f32) + b4[...], 0.0)
        acc = jnp.zeros((_R, 40), f32) + bfw[...]
        for s in range(6):
            acc = acc + jnp.dot(h[s * _R:(s + 1) * _R, :], fw[s],
                                preferred_element_type=f32)
        hw = jnp.maximum(acc, 0.0)

        xs = jnp.concatenate(
            [xb[:, 312 + s * 10:312 + (s + 1) * 10] for s in range(10)], axis=0)
        g = jnp.maximum(jnp.dot(xs, s1[...], preferred_element_type=f32) + c1[...], 0.0)
        g = jnp.maximum(jnp.dot(g, s2[...], preferred_element_type=f32) + c2[...], 0.0)
        g = jnp.maximum(jnp.dot(g, s3[...], preferred_element_type=f32) + c3[...], 0.0)
        acc2 = jnp.zeros((_R, 40), f32) + bfs[...]
        for s in range(10):
            acc2 = acc2 + jnp.dot(g[s * _R:(s + 1) * _R, :], fs[s],
                                  preferred_element_type=f32)
        hs = jnp.maximum(acc2, 0.0)

        out = jnp.concatenate(
            [hw, hs, xb[:, 412:431], jnp.ones((_R, 1), f32),
             jnp.zeros((_R, 28), f32)], axis=1)
        out_ref[t] = out


def _full_spec(a):
    nd = a.ndim
    return pl.BlockSpec(a.shape, lambda *_, _nd=nd: (0,) * _nd)


def _cnn_call(x, weights):
    in_specs = [pl.BlockSpec((_R, T, 431), lambda nb: (nb, 0, 0))]
    in_specs += [_full_spec(w) for w in weights]
    return pl.pallas_call(
        _cnn_body,
        grid=(_NB,),
        in_specs=in_specs,
        out_specs=pl.BlockSpec((T, _R, _W0), lambda nb: (0, nb, 0)),
        out_shape=jax.ShapeDtypeStruct((T, N0, _W0), jnp.float32),
    )(x, *weights)


def _make_agg(w, n_dst, n_chunks, ch):
    rpt = n_dst // 16
    nz = rpt // 16
    mesh = plsc.VectorSubcoreMesh(core_axis_name="c", subcore_axis_name="s",
                                  num_cores=2, num_subcores=16)

    def body(table, srci, dsti, out, srcv, dstv, rows, zbuf, shared, sem):
        c = lax.axis_index("c")
        s = lax.axis_index("s")
        wid = s * 2 + c
        pltpu.sync_copy(srci.at[wid], srcv)
        pltpu.sync_copy(dsti.at[wid], dstv)
        zero = jnp.zeros((16,), jnp.float32)
        for r in range(16):
            for q in range(w // 16):
                zbuf[r, pl.ds(q * 16, 16)] = zero
        for k in range(nz):
            pltpu.sync_copy(zbuf, shared.at[pl.ds(s * rpt + k * 16, 16)])
        plsc.subcore_barrier()
        for j in range(n_chunks):
            pltpu.async_copy(table.at[srcv.at[j]], rows, sem).wait()
            pltpu.sync_copy(rows, shared.at[dstv.at[j]], add=True)
        plsc.subcore_barrier()
        pltpu.sync_copy(shared.at[pl.ds(s * rpt, rpt)],
                        out.at[c, pl.ds(s * rpt, rpt)])

    return pl.kernel(
        body,
        out_type=jax.ShapeDtypeStruct((2, n_dst, w), jnp.float32),
        mesh=mesh,
        scratch_types=[
            pltpu.VMEM((n_chunks, ch), jnp.int32),
            pltpu.VMEM((n_chunks, ch), jnp.int32),
            pltpu.VMEM((ch, w), jnp.float32),
            pltpu.VMEM((16, w), jnp.float32),
            pltpu.VMEM_SHARED((n_dst, w), jnp.float32),
            pltpu.SemaphoreType.DMA,
        ],
    )


def _mm0_body(p_ref, hp_ref, ws_ref, wn_ref, out_ref):
    f32 = jnp.float32
    ssum = p_ref[0] + p_ref[1]
    inv = 1.0 / jnp.maximum(ssum[:, 99:100], 1.0)
    z = (jnp.dot(hp_ref[0], ws_ref[...], preferred_element_type=f32)
         + jnp.dot(ssum * inv, wn_ref[...], preferred_element_type=f32))
    out_ref[...] = jnp.maximum(z, 0.0)


def _mm0_call(p, hfull, ws, wn, t):
    return pl.pallas_call(
        _mm0_body,
        grid=(N1 // _RB,),
        in_specs=[
            pl.BlockSpec((2, _RB, _W0), lambda nb: (0, nb, 0)),
            pl.BlockSpec((1, _RB, _W0), lambda nb, _t=t: (_t, nb, 0)),
            _full_spec(ws), _full_spec(wn),
        ],
        out_specs=pl.BlockSpec((_RB, H), lambda nb: (nb, 0)),
        out_shape=jax.ShapeDtypeStruct((N1, H), jnp.float32),
    )(p, hfull, ws, wn)


def _mm1_body(p_ref, c_ref, zp_ref, ws_ref, wn_ref, wb_ref, out_ref):
    f32 = jnp.float32
    ssum = p_ref[0] + p_ref[1]
    cnt = c_ref[0, :, 0:1] + c_ref[1, :, 0:1]
    inv = 1.0 / jnp.maximum(cnt, 1.0)
    z = (jnp.dot(zp_ref[...], ws_ref[...], preferred_element_type=f32)
         + jnp.dot(ssum * inv, wn_ref[...], preferred_element_type=f32)
         + wb_ref[...])
    out_ref[...] = jnp.maximum(z, 0.0)


def _mm1_call(p, cntp, z1p, ws, wn, wb):
    return pl.pallas_call(
        _mm1_body,
        grid=(1,),
        in_specs=[
            pl.BlockSpec((2, N2, H), lambda i: (0, 0, 0)),
            pl.BlockSpec((2, N2, H), lambda i: (0, 0, 0)),
            pl.BlockSpec((N2, H), lambda i: (0, 0)),
            _full_spec(ws), _full_spec(wn), _full_spec(wb),
        ],
        out_specs=pl.BlockSpec((N2, H), lambda i: (0, 0)),
        out_shape=jax.ShapeDtypeStruct((N2, H), jnp.float32),
    )(p, cntp, z1p, ws, wn, wb)


def _lstm_body(z0, z1, z2, z3, z4, y_ref, a_ref, ay_ref, bm_ref, bias_ref,
               r1t_ref, r1b_ref, r2t_ref, r2b_ref, out_ref):
    f32 = jnp.float32
    a = a_ref[...]
    ay = ay_ref[...]
    bm = bm_ref[...]
    bias = bias_ref[...]
    hh = jnp.zeros((N2, H), f32)
    cc = jnp.zeros((N2, H), f32)
    zs = [z0, z1, z2, z3, z4]
    for t in range(T):
        gates = (jnp.dot(zs[t][...], a, preferred_element_type=f32)
                 + jnp.dot(hh, bm, preferred_element_type=f32) + bias)
        if t > 0:
            gates = gates + y_ref[:, t - 1:t] * ay
        ig = jax.nn.sigmoid(gates[:, :H])
        fg = jax.nn.sigmoid(gates[:, H:2 * H])
        gg = jnp.tanh(gates[:, 2 * H:3 * H])
        og = jax.nn.sigmoid(gates[:, 3 * H:])
        cc = fg * cc + ig * gg
        hh = og * jnp.tanh(cc)
    z = jnp.maximum(jnp.dot(hh, r1t_ref[...], preferred_element_type=f32)
                    + r1b_ref[...], 0.0)
    out_ref[...] = jnp.dot(z, r2t_ref[...], preferred_element_type=f32) + r2b_ref[...]


def _lstm_call(z2s, y, lw):
    args = list(z2s) + [y] + list(lw)
    return pl.pallas_call(
        _lstm_body,
        grid=(1,),
        in_specs=[_full_spec(a2) for a2 in args],
        out_specs=pl.BlockSpec((N2, H), lambda i: (0, 0)),
        out_shape=jax.ShapeDtypeStruct((N2, H), jnp.float32),
    )(*args)


def kernel(x, y, wc1, wb1, wc2, wb2, wc3, wb3, wc4, wb4, wfc, wfcb,
           sc1, sb1, sc2, sb2, sc3, sb3, sfc, sfcb,
           g0s, g0n, g0b, g1s, g1n, g1b, lih, lhh, lbi, lbh,
           r1, r1b, r2, r2b, edge_src0, edge_dst0, edge_src1, edge_dst1):
    f32 = jnp.float32

    m1 = _conv_matrix(wc1, 52, 44)
    b1 = jnp.repeat(wb1, 44)[None]
    m2 = _pool_matrix(8, 44) @ _conv_matrix(wc2, 22, 20)
    b2 = jnp.repeat(wb2, 20)[None]
    m3 = _pool_matrix(12, 20) @ _conv_matrix(wc3, 10, 8)
    b3 = jnp.repeat(wb3, 8)[None]
    m4 = _pool_matrix(16, 8) @ _conv_matrix(wc4, 4, 2)
    b4 = jnp.repeat(wb4, 2)[None]
    p4 = _pool_matrix(20, 2)
    wfct = wfc.T
    fw = jnp.stack([p4 @ wfct[s * 20:(s + 1) * 20] for s in range(6)])
    bfw = wfcb[None]
    s1m = _conv_matrix(sc1, 10, 8)
    c1 = jnp.repeat(sb1, 8)[None]
    s2m = _pool_matrix(4, 8) @ _conv_matrix(sc2, 4, 2)
    c2 = jnp.repeat(sb2, 2)[None]
    s3m = _conv_matrix(sc3, 2, 1)
    c3 = sb3[None]
    sfct = sfc.T
    fs = jnp.stack([sfct[s * 12:(s + 1) * 12] for s in range(10)])
    bfs = sfcb[None]
    cnn_ws = [m1, b1, m2, b2, m3, b3, m4, b4, fw, bfw,
              s1m, c1, s2m, c2, s3m, c3, fs, bfs]

    g0s_pad = jnp.zeros((_W0, H), f32).at[:99].set(g0s).at[99].set(g0b)
    g0n_pad = jnp.zeros((_W0, H), f32).at[:99].set(g0n)
    g1b2 = g1b[None]

    lw = [lih[:, :H].T, lih[:, H][None], lhh.T, (lbi + lbh)[None],
          r1.T, r1b[None],
          jnp.zeros((H // 2, H), f32).at[:, 0].set(r2[0]),
          jnp.zeros((1, H), f32).at[0, 0].set(r2b[0])]

    dst0r = edge_dst0.reshape(_NW, _NC0, _CH0)
    src1r = edge_src1.reshape(_NW, _NC1, _CH1)
    dst1r = edge_dst1.reshape(_NW, _NC1, _CH1)

    agg0 = _make_agg(_W0, N1, _NC0, _CH0)
    agg1 = _make_agg(_W0, N2, _NC1, _CH1)

    hfull = _cnn_call(x, cnn_ws)
    table0 = hfull.reshape(T * N0, _W0)

    ones_tab = jnp.ones((8, _W0), f32)
    zsrc = jnp.zeros((_NW, _NC1, _CH1), jnp.int32)
    cntp = agg1(ones_tab, zsrc, dst1r)

    z2s = []
    for t in range(T):
        src0r = (edge_src0 + t * N0).reshape(_NW, _NC0, _CH0)
        p0 = agg0(table0, src0r, dst0r)
        z1 = _mm0_call(p0, hfull, g0s_pad, g0n_pad, t)
        p1 = agg1(z1, src1r, dst1r)
        z2 = _mm1_call(p1, cntp, z1, g1s, g1n, g1b2)
        z2s.append(z2)

    out = _lstm_call(z2s, y, lw)
    return out[:, :1]


# speedup vs baseline: 4.1858x; 1.1970x over previous
import jax
import jax.numpy as jnp
import numpy as np
from jax import lax
from jax.experimental import pallas as pl
from jax.experimental.pallas import tpu as pltpu
from jax.experimental.pallas import tpu_sc as plsc

N0, N1, N2 = 50176, 7168, 1024
T = 5
H = 128

_R = 256
_NB = N0 // _R
_RB = 896
_NW = 32

_E0, _NC0, _CH0 = 43008, 12, 112
_E1, _NC1, _CH1 = 6144, 2, 96
_W0 = 128


def _conv_matrix(w, l_in, l_out):
    c_out, c_in, k = w.shape
    cols = jnp.stack([jnp.pad(w, ((0, 0), (0, 0), (lo, l_in - k - lo)))
                      for lo in range(l_out)], axis=2)
    return cols.transpose(1, 3, 0, 2).reshape(c_in * l_in, c_out * l_out)


def _pool_matrix(c, l_in):
    l_out = l_in // 2
    p = np.zeros((c * l_in, c * l_out), np.float32)
    for ci in range(c):
        for l in range(l_in):
            p[ci * l_in + l, ci * l_out + l // 2] = 0.5
    return jnp.asarray(p)


def _cnn_body(x_ref, m1, b1, m2, b2, m3, b3, m4, b4, fw, bfw,
              s1, c1, s2, c2, s3, c3, fs, bfs, out_ref):
    f32 = jnp.float32
    for t in range(T):
        xb = x_ref[:, t * 431:(t + 1) * 431]

        xw = jnp.concatenate([xb[:, s * 52:(s + 1) * 52] for s in range(6)],
                             axis=0)
        h = jnp.maximum(jnp.dot(xw, m1[...], preferred_element_type=f32) + b1[...], 0.0)
        h = jnp.maximum(jnp.dot(h, m2[...], preferred_element_type=f32) + b2[...], 0.0)
        h = jnp.maximum(jnp.dot(h, m3[...], preferred_element_type=f32) + b3[...], 0.0)
        h = jnp.maximum(jnp.dot(h, m4[...], preferred_element_type=f32) + b4[...], 0.0)
        acc = jnp.zeros((_R, 40), f32) + bfw[...]
        for s in range(6):
            acc = acc + jnp.dot(h[s * _R:(s + 1) * _R, :], fw[s],
                                preferred_element_type=f32)
        hw = jnp.maximum(acc, 0.0)

        xs = jnp.concatenate(
            [xb[:, 312 + s * 10:312 + (s + 1) * 10] for s in range(10)], axis=0)
        g = jnp.maximum(jnp.dot(xs, s1[...], preferred_element_type=f32) + c1[...], 0.0)
        g = jnp.maximum(jnp.dot(g, s2[...], preferred_element_type=f32) + c2[...], 0.0)
        g = jnp.maximum(jnp.dot(g, s3[...], preferred_element_type=f32) + c3[...], 0.0)
        acc2 = jnp.zeros((_R, 40), f32) + bfs[...]
        for s in range(10):
            acc2 = acc2 + jnp.dot(g[s * _R:(s + 1) * _R, :], fs[s],
                                  preferred_element_type=f32)
        hs = jnp.maximum(acc2, 0.0)

        out = jnp.concatenate(
            [hw, hs, xb[:, 412:431], jnp.ones((_R, 1), f32),
             jnp.zeros((_R, 28), f32)], axis=1)
        out_ref[t] = out


def _full_spec(a):
    nd = a.ndim
    return pl.BlockSpec(a.shape, lambda *_, _nd=nd: (0,) * _nd)


def _cnn_call(x2, weights):
    in_specs = [pl.BlockSpec((_R, T * 431), lambda nb: (nb, 0))]
    in_specs += [_full_spec(w) for w in weights]
    return pl.pallas_call(
        _cnn_body,
        grid=(_NB,),
        in_specs=in_specs,
        out_specs=pl.BlockSpec((T, _R, _W0), lambda nb: (0, nb, 0)),
        out_shape=jax.ShapeDtypeStruct((T, N0, _W0), jnp.float32),
    )(x2, *weights)


def _make_agg(w, n_dst, n_chunks, ch):
    rpt = n_dst // 16
    nz = rpt // 16
    mesh = plsc.VectorSubcoreMesh(core_axis_name="c", subcore_axis_name="s",
                                  num_cores=2, num_subcores=16)

    def body(table, srci, dsti, out, srcv, dstv, rows, zbuf, shared, sem):
        c = lax.axis_index("c")
        s = lax.axis_index("s")
        wid = s * 2 + c
        pltpu.sync_copy(srci.at[wid], srcv)
        pltpu.sync_copy(dsti.at[wid], dstv)
        zero = jnp.zeros((16,), jnp.float32)
        for r in range(16):
            for q in range(w // 16):
                zbuf[r, pl.ds(q * 16, 16)] = zero
        for k in range(nz):
            pltpu.sync_copy(zbuf, shared.at[pl.ds(s * rpt + k * 16, 16)])
        plsc.subcore_barrier()
        for j in range(n_chunks):
            pltpu.async_copy(table.at[srcv.at[j]], rows, sem).wait()
            pltpu.sync_copy(rows, shared.at[dstv.at[j]], add=True)
        plsc.subcore_barrier()
        pltpu.sync_copy(shared.at[pl.ds(s * rpt, rpt)],
                        out.at[c, pl.ds(s * rpt, rpt)])

    return pl.kernel(
        body,
        out_type=jax.ShapeDtypeStruct((2, n_dst, w), jnp.float32),
        mesh=mesh,
        scratch_types=[
            pltpu.VMEM((n_chunks, ch), jnp.int32),
            pltpu.VMEM((n_chunks, ch), jnp.int32),
            pltpu.VMEM((ch, w), jnp.float32),
            pltpu.VMEM((16, w), jnp.float32),
            pltpu.VMEM_SHARED((n_dst, w), jnp.float32),
            pltpu.SemaphoreType.DMA,
        ],
    )


def _make_cnt(w, n_dst, n_chunks, ch):
    rpt = n_dst // 16
    nz = rpt // 16
    mesh = plsc.VectorSubcoreMesh(core_axis_name="c", subcore_axis_name="s",
                                  num_cores=2, num_subcores=16)

    def body(dsti, out, dstv, ones_rows, zbuf, shared):
        c = lax.axis_index("c")
        s = lax.axis_index("s")
        wid = s * 2 + c
        pltpu.sync_copy(dsti.at[wid], dstv)
        zero = jnp.zeros((16,), jnp.float32)
        one = jnp.ones((16,), jnp.float32)
        for r in range(16):
            for q in range(w // 16):
                zbuf[r, pl.ds(q * 16, 16)] = zero
        for r in range(ch):
            for q in range(w // 16):
                ones_rows[r, pl.ds(q * 16, 16)] = one
        for k in range(nz):
            pltpu.sync_copy(zbuf, shared.at[pl.ds(s * rpt + k * 16, 16)])
        plsc.subcore_barrier()
        for j in range(n_chunks):
            pltpu.sync_copy(ones_rows, shared.at[dstv.at[j]], add=True)
        plsc.subcore_barrier()
        pltpu.sync_copy(shared.at[pl.ds(s * rpt, rpt)],
                        out.at[c, pl.ds(s * rpt, rpt)])

    return pl.kernel(
        body,
        out_type=jax.ShapeDtypeStruct((2, n_dst, w), jnp.float32),
        mesh=mesh,
        scratch_types=[
            pltpu.VMEM((n_chunks, ch), jnp.int32),
            pltpu.VMEM((ch, w), jnp.float32),
            pltpu.VMEM((16, w), jnp.float32),
            pltpu.VMEM_SHARED((n_dst, w), jnp.float32),
        ],
    )


def _mm0_body(p_ref, hp_ref, ws_ref, wn_ref, out_ref):
    f32 = jnp.float32
    ssum = p_ref[0] + p_ref[1]
    inv = 1.0 / jnp.maximum(ssum[:, 99:100], 1.0)
    z = (jnp.dot(hp_ref[0], ws_ref[...], preferred_element_type=f32)
         + jnp.dot(ssum * inv, wn_ref[...], preferred_element_type=f32))
    out_ref[...] = jnp.maximum(z, 0.0)


def _mm0_call(p, hfull, ws, wn, t):
    return pl.pallas_call(
        _mm0_body,
        grid=(N1 // _RB,),
        in_specs=[
            pl.BlockSpec((2, _RB, _W0), lambda nb: (0, nb, 0)),
            pl.BlockSpec((1, _RB, _W0), lambda nb, _t=t: (_t, nb, 0)),
            _full_spec(ws), _full_spec(wn),
        ],
        out_specs=pl.BlockSpec((_RB, H), lambda nb: (nb, 0)),
        out_shape=jax.ShapeDtypeStruct((N1, H), jnp.float32),
    )(p, hfull, ws, wn)


def _mm1_body(p_ref, c_ref, zp_ref, ws_ref, wn_ref, wb_ref, out_ref):
    f32 = jnp.float32
    ssum = p_ref[0] + p_ref[1]
    cnt = c_ref[0, :, 0:1] + c_ref[1, :, 0:1]
    inv = 1.0 / jnp.maximum(cnt, 1.0)
    z = (jnp.dot(zp_ref[...], ws_ref[...], preferred_element_type=f32)
         + jnp.dot(ssum * inv, wn_ref[...], preferred_element_type=f32)
         + wb_ref[...])
    out_ref[...] = jnp.maximum(z, 0.0)


def _mm1_call(p, cntp, z1p, ws, wn, wb):
    return pl.pallas_call(
        _mm1_body,
        grid=(1,),
        in_specs=[
            pl.BlockSpec((2, N2, H), lambda i: (0, 0, 0)),
            pl.BlockSpec((2, N2, H), lambda i: (0, 0, 0)),
            pl.BlockSpec((N2, H), lambda i: (0, 0)),
            _full_spec(ws), _full_spec(wn), _full_spec(wb),
        ],
        out_specs=pl.BlockSpec((N2, H), lambda i: (0, 0)),
        out_shape=jax.ShapeDtypeStruct((N2, H), jnp.float32),
    )(p, cntp, z1p, ws, wn, wb)


def _lstm_body(z0, z1, z2, z3, z4, y_ref, a_ref, ay_ref, bm_ref, bias_ref,
               r1t_ref, r1b_ref, r2t_ref, r2b_ref, out_ref):
    f32 = jnp.float32
    a = a_ref[...]
    ay = ay_ref[...]
    bm = bm_ref[...]
    bias = bias_ref[...]
    hh = jnp.zeros((N2, H), f32)
    cc = jnp.zeros((N2, H), f32)
    zs = [z0, z1, z2, z3, z4]
    for t in range(T):
        gates = (jnp.dot(zs[t][...], a, preferred_element_type=f32)
                 + jnp.dot(hh, bm, preferred_element_type=f32) + bias)
        if t > 0:
            gates = gates + y_ref[:, t - 1:t] * ay
        ig = jax.nn.sigmoid(gates[:, :H])
        fg = jax.nn.sigmoid(gates[:, H:2 * H])
        gg = jnp.tanh(gates[:, 2 * H:3 * H])
        og = jax.nn.sigmoid(gates[:, 3 * H:])
        cc = fg * cc + ig * gg
        hh = og * jnp.tanh(cc)
    z = jnp.maximum(jnp.dot(hh, r1t_ref[...], preferred_element_type=f32)
                    + r1b_ref[...], 0.0)
    out_ref[...] = jnp.dot(z, r2t_ref[...], preferred_element_type=f32) + r2b_ref[...]


def _lstm_call(z2s, y, lw):
    args = list(z2s) + [y] + list(lw)
    return pl.pallas_call(
        _lstm_body,
        grid=(1,),
        in_specs=[_full_spec(a2) for a2 in args],
        out_specs=pl.BlockSpec((N2, H), lambda i: (0, 0)),
        out_shape=jax.ShapeDtypeStruct((N2, H), jnp.float32),
    )(*args)


def kernel(x, y, wc1, wb1, wc2, wb2, wc3, wb3, wc4, wb4, wfc, wfcb,
           sc1, sb1, sc2, sb2, sc3, sb3, sfc, sfcb,
           g0s, g0n, g0b, g1s, g1n, g1b, lih, lhh, lbi, lbh,
           r1, r1b, r2, r2b, edge_src0, edge_dst0, edge_src1, edge_dst1):
    f32 = jnp.float32

    m1 = _conv_matrix(wc1, 52, 44)
    b1 = jnp.repeat(wb1, 44)[None]
    m2 = _pool_matrix(8, 44) @ _conv_matrix(wc2, 22, 20)
    b2 = jnp.repeat(wb2, 20)[None]
    m3 = _pool_matrix(12, 20) @ _conv_matrix(wc3, 10, 8)
    b3 = jnp.repeat(wb3, 8)[None]
    m4 = _pool_matrix(16, 8) @ _conv_matrix(wc4, 4, 2)
    b4 = jnp.repeat(wb4, 2)[None]
    p4 = _pool_matrix(20, 2)
    wfct = wfc.T
    fw = jnp.stack([p4 @ wfct[s * 20:(s + 1) * 20] for s in range(6)])
    bfw = wfcb[None]
    s1m = _conv_matrix(sc1, 10, 8)
    c1 = jnp.repeat(sb1, 8)[None]
    s2m = _pool_matrix(4, 8) @ _conv_matrix(sc2, 4, 2)
    c2 = jnp.repeat(sb2, 2)[None]
    s3m = _conv_matrix(sc3, 2, 1)
    c3 = sb3[None]
    sfct = sfc.T
    fs = jnp.stack([sfct[s * 12:(s + 1) * 12] for s in range(10)])
    bfs = sfcb[None]
    cnn_ws = [m1, b1, m2, b2, m3, b3, m4, b4, fw, bfw,
              s1m, c1, s2m, c2, s3m, c3, fs, bfs]

    g0s_pad = jnp.zeros((_W0, H), f32).at[:99].set(g0s).at[99].set(g0b)
    g0n_pad = jnp.zeros((_W0, H), f32).at[:99].set(g0n)
    g1b2 = g1b[None]

    lw = [lih[:, :H].T, lih[:, H][None], lhh.T, (lbi + lbh)[None],
          r1.T, r1b[None],
          jnp.zeros((H // 2, H), f32).at[:, 0].set(r2[0]),
          jnp.zeros((1, H), f32).at[0, 0].set(r2b[0])]

    dst0r = edge_dst0.reshape(_NW, _NC0, _CH0)
    src1r = edge_src1.reshape(_NW, _NC1, _CH1)
    dst1r = edge_dst1.reshape(_NW, _NC1, _CH1)

    agg0 = _make_agg(_W0, N1, _NC0, _CH0)
    agg1 = _make_agg(_W0, N2, _NC1, _CH1)

    hfull = _cnn_call(x.reshape(N0, T * 431), cnn_ws)
    table0 = hfull.reshape(T * N0, _W0)

    cnt1k = _make_cnt(_W0, N2, _NC1, _CH1)
    cntp = cnt1k(dst1r)

    z2s = []
    for t in range(T):
        src0r = (edge_src0 + t * N0).reshape(_NW, _NC0, _CH0)
        p0 = agg0(table0, src0r, dst0r)
        z1 = _mm0_call(p0, hfull, g0s_pad, g0n_pad, t)
        p1 = agg1(z1, src1r, dst1r)
        z2 = _mm1_call(p1, cntp, z1, g1s, g1n, g1b2)
        z2s.append(z2)

    out = _lstm_call(z2s, y, lw)
    return out[:, :1]
